# baseline (speedup 1.0000x reference)
import functools

import jax
import jax.numpy as jnp
from jax import lax
from jax.experimental import pallas as pl
from jax.experimental.pallas import tpu as pltpu
from jax.experimental.pallas import tpu_sc as plsc

B, C, H, W = 32, 768, 32, 32
N = H * W
R = B * C
K = 205
ALPHA = 0.7

NC, NS, L = 2, 16, 16
NW = NC * NS
RPW = R // NW
GROUP = 16
NGRP = RPW // GROUP
NBITS = 12
NW2 = N // 2

_DNUMS = lax.GatherDimensionNumbers(
    offset_dims=(), collapsed_slice_dims=(0,), start_index_map=(0,))


def _permute(v, p):
    return lax.gather(v, p[:, None], dimension_numbers=_DNUMS,
                      slice_sizes=(1,),
                      mode=lax.GatherScatterMode.PROMISE_IN_BOUNDS)


def _kernel_body(x_hbm, out_hbm, xbuf0, xbuf1, outbuf, sem0, sem1):
    wid = lax.axis_index("s") * NC + lax.axis_index("c")
    zero = jnp.zeros((L,), jnp.int32)
    one = jnp.ones((L,), jnp.int32)
    fzero = jnp.zeros((L,), jnp.float32)
    bzero = jnp.zeros((2 * L,), jnp.bfloat16)
    bone = jnp.ones((2 * L,), jnp.bfloat16)
    lanes = lax.iota(jnp.int32, L)
    perms = [lanes ^ sh for sh in (8, 4, 2, 1)]

    def allsum(v):
        for p in perms:
            v = v + _permute(v, p)
        return v

    def u2bits(u):
        return jnp.where(u >= 32768, u - 32768, 65535 - u)

    def u2f32(u):
        return plsc.bitcast(u2bits(u) << 16, jnp.float32)

    def u2bf(u):
        b = u2bits(u)
        return plsc.bitcast(b | (b << 16), jnp.bfloat16)

    GN2 = GROUP * NW2

    def copy_in(g, xb, sem):
        row0 = wid * RPW + g * GROUP
        return pltpu.make_async_copy(
            x_hbm.at[pl.ds(row0 * NW2, GN2)], xb, sem)

    def process(xb, g):
        def row_body(r, ovec):
            rbase = r * NW2

            t1v, t2v = zero, zero
            for i in range(NBITS):
                bitc = 32768 >> i
                cand1 = t1v + bitc
                cand2 = t2v + bitc
                cv1 = u2bf(cand1)
                cv2 = u2bf(65535 - cand2)

                def cbody(j, c, cv1=cv1, cv2=cv2):
                    c1a, c1b, c2a, c2b = c
                    for u in range(4):
                        v = plsc.bitcast(
                            xb[pl.ds(rbase + (j * 4 + u) * L, L)],
                            jnp.bfloat16)
                        i1 = jnp.where(v >= cv1, bone, bzero)
                        i2 = jnp.where(v <= cv2, bone, bzero)
                        if u % 2 == 0:
                            c1a = c1a + i1
                            c2a = c2a + i2
                        else:
                            c1b = c1b + i1
                            c2b = c2b + i2
                    return c1a, c1b, c2a, c2b

                c1a, c1b, c2a, c2b = lax.fori_loop(
                    0, NW2 // (4 * L), cbody, (bzero, bzero, bzero, bzero))
                u1a, u1b = plsc.unpack(c1a + c1b,
                                       format=plsc.PackFormat.INTERLEAVED)
                u2a, u2b = plsc.unpack(c2a + c2b,
                                       format=plsc.PackFormat.INTERLEAVED)
                n1 = allsum(u1a + u1b)
                n2 = allsum(u2a + u2b)
                t1v = jnp.where(n1 >= float(K), cand1, t1v)
                t2v = jnp.where(n2 >= float(K), cand2, t2v)
            t1 = t1v
            bot = 65535 - t2v

            val_top = 0.5 * (u2f32(t1) + u2f32(t1 + 15))
            val_bot = 0.5 * (u2f32(bot - 15) + u2f32(bot))
            ub = u2f32(t1 + 15)
            lb = u2f32(bot - 15)

            def fbody(j, c):
                cg, sg, cl, sl = c
                for u in range(4):
                    v = plsc.bitcast(
                        xb[pl.ds(rbase + (j * 4 + u) * L, L)],
                        jnp.bfloat16)
                    a, b = plsc.unpack(v, format=plsc.PackFormat.INTERLEAVED)
                    for xv in (a, b):
                        m1 = xv > ub
                        m2 = xv < lb
                        cg = cg + jnp.where(m1, one, zero)
                        sg = sg + jnp.where(m1, xv, fzero)
                        cl = cl + jnp.where(m2, one, zero)
                        sl = sl + jnp.where(m2, xv, fzero)
                return cg, sg, cl, sl

            cg, sg, cl, sl = lax.fori_loop(
                0, NW2 // (4 * L), fbody, (zero, fzero, zero, fzero))

            ng = float(K) - allsum(cg.astype(jnp.float32))
            nl = float(K) - allsum(cl.astype(jnp.float32))
            sgv = allsum(sg)
            slv = allsum(sl)
            top_sum = sgv + ng * val_top
            bot_sum = slv + nl * val_bot
            outv = top_sum * (1.0 / (2 * K)) + bot_sum * (ALPHA / (2 * K))
            return jnp.where(lanes == r, outv, ovec)

        ovec = lax.fori_loop(0, GROUP, row_body, fzero)
        outbuf[pl.ds(g * GROUP, GROUP)] = ovec

    copy_in(0, xbuf0, sem0).start()

    def group_body(gg, carry):
        g0 = 2 * gg
        copy_in(g0, xbuf0, sem0).wait()
        copy_in(g0 + 1, xbuf1, sem1).start()
        process(xbuf0, g0)
        copy_in(g0 + 1, xbuf1, sem1).wait()

        @pl.when(gg + 1 < NGRP // 2)
        def _():
            copy_in(g0 + 2, xbuf0, sem0).start()

        process(xbuf1, g0 + 1)
        return carry

    lax.fori_loop(0, NGRP // 2, group_body, 0)
    pltpu.sync_copy(outbuf, out_hbm.at[pl.ds(wid * RPW, RPW)])


@jax.jit
def kernel(input):
    xb16 = input.astype(jnp.bfloat16).reshape(R * NW2, 2)
    x = jax.lax.bitcast_convert_type(xb16, jnp.int32)
    mesh = plsc.VectorSubcoreMesh(
        core_axis_name="c", subcore_axis_name="s",
        num_cores=NC, num_subcores=NS)
    out = pl.kernel(
        _kernel_body,
        out_type=jax.ShapeDtypeStruct((R,), jnp.float32),
        mesh=mesh,
        compiler_params=pltpu.CompilerParams(needs_layout_passes=False),
        scratch_types=[
            pltpu.VMEM((GROUP * NW2,), jnp.int32),
            pltpu.VMEM((GROUP * NW2,), jnp.int32),
            pltpu.VMEM((RPW,), jnp.float32),
            pltpu.SemaphoreType.DMA,
            pltpu.SemaphoreType.DMA,
        ],
    )(x)
    return out.reshape(B, C)


# speedup vs baseline: 6.7727x; 6.7727x over previous
import functools

import jax
import jax.numpy as jnp
from jax import lax
from jax.experimental import pallas as pl
from jax.experimental.pallas import tpu as pltpu
from jax.experimental.pallas import tpu_sc as plsc

B, C, H, W = 32, 768, 32, 32
N = H * W
R = B * C
K = 205
ALPHA = 0.7

NC, NS, L = 2, 16, 16
NW = NC * NS
RPW = R // NW
GROUP = 16
NGRP = RPW // GROUP
NBITS = 12
NW2 = N // 2

_DNUMS = lax.GatherDimensionNumbers(
    offset_dims=(), collapsed_slice_dims=(0,), start_index_map=(0,))


def _permute(v, p):
    return lax.gather(v, p[:, None], dimension_numbers=_DNUMS,
                      slice_sizes=(1,),
                      mode=lax.GatherScatterMode.PROMISE_IN_BOUNDS)


def _kernel_body(x_hbm, out_hbm, xbuf0, xbuf1, outbuf, sem0, sem1):
    wid = lax.axis_index("s") * NC + lax.axis_index("c")
    zero = jnp.zeros((L,), jnp.int32)
    one = jnp.ones((L,), jnp.int32)
    fzero = jnp.zeros((L,), jnp.float32)
    bzero = jnp.zeros((2 * L,), jnp.bfloat16)
    bone = jnp.ones((2 * L,), jnp.bfloat16)
    lanes = lax.iota(jnp.int32, L)
    perms = [lanes ^ sh for sh in (8, 4, 2, 1)]

    def allsum(v):
        for p in perms:
            v = v + _permute(v, p)
        return v

    def u2bits(u):
        return jnp.where(u >= 32768, u - 32768, 65535 - u)

    def u2f32(u):
        return plsc.bitcast(u2bits(u) << 16, jnp.float32)

    def u2bf(u):
        b = u2bits(u)
        return plsc.bitcast(b | (b << 16), jnp.bfloat16)

    GN2 = GROUP * NW2

    def copy_in(g, xb, sem):
        row0 = wid * RPW + g * GROUP
        return pltpu.make_async_copy(
            x_hbm.at[pl.ds(row0 * N, GROUP * N)], xb, sem)

    def process(xb, g):
        def row_body(r, ovec):
            rbase = r * N

            t1v, t2v = zero, zero
            for i in range(NBITS):
                bitc = 32768 >> i
                cand1 = t1v + bitc
                cand2 = t2v + bitc
                cv1 = u2bf(cand1)
                cv2 = u2bf(65535 - cand2)

                def cbody(j, c, cv1=cv1, cv2=cv2):
                    c1a, c1b, c2a, c2b = c
                    for u in range(4):
                        v = xb[pl.ds(rbase + (j * 4 + u) * 2 * L, 2 * L)]
                        i1 = jnp.where(v >= cv1, bone, bzero)
                        i2 = jnp.where(v <= cv2, bone, bzero)
                        if u % 2 == 0:
                            c1a = c1a + i1
                            c2a = c2a + i2
                        else:
                            c1b = c1b + i1
                            c2b = c2b + i2
                    return c1a, c1b, c2a, c2b

                c1a, c1b, c2a, c2b = lax.fori_loop(
                    0, NW2 // (4 * L), cbody, (bzero, bzero, bzero, bzero))
                u1a, u1b = plsc.unpack(c1a + c1b,
                                       format=plsc.PackFormat.INTERLEAVED)
                u2a, u2b = plsc.unpack(c2a + c2b,
                                       format=plsc.PackFormat.INTERLEAVED)
                n1 = allsum(u1a + u1b)
                n2 = allsum(u2a + u2b)
                t1v = jnp.where(n1 >= float(K), cand1, t1v)
                t2v = jnp.where(n2 >= float(K), cand2, t2v)
            t1 = t1v
            bot = 65535 - t2v

            val_top = 0.5 * (u2f32(t1) + u2f32(t1 + 15))
            val_bot = 0.5 * (u2f32(bot - 15) + u2f32(bot))
            ub = u2f32(t1 + 15)
            lb = u2f32(bot - 15)

            def fbody(j, c):
                cg, sg, cl, sl = c
                for u in range(4):
                    v = xb[pl.ds(rbase + (j * 4 + u) * 2 * L, 2 * L)]
                    a, b = plsc.unpack(v, format=plsc.PackFormat.INTERLEAVED)
                    for xv in (a, b):
                        m1 = xv > ub
                        m2 = xv < lb
                        cg = cg + jnp.where(m1, one, zero)
                        sg = sg + jnp.where(m1, xv, fzero)
                        cl = cl + jnp.where(m2, one, zero)
                        sl = sl + jnp.where(m2, xv, fzero)
                return cg, sg, cl, sl

            cg, sg, cl, sl = lax.fori_loop(
                0, NW2 // (4 * L), fbody, (zero, fzero, zero, fzero))

            ng = float(K) - allsum(cg.astype(jnp.float32))
            nl = float(K) - allsum(cl.astype(jnp.float32))
            sgv = allsum(sg)
            slv = allsum(sl)
            top_sum = sgv + ng * val_top
            bot_sum = slv + nl * val_bot
            outv = top_sum * (1.0 / (2 * K)) + bot_sum * (ALPHA / (2 * K))
            return jnp.where(lanes == r, outv, ovec)

        ovec = lax.fori_loop(0, GROUP, row_body, fzero)
        outbuf[pl.ds(g * GROUP, GROUP)] = ovec

    copy_in(0, xbuf0, sem0).start()

    def group_body(gg, carry):
        g0 = 2 * gg
        copy_in(g0, xbuf0, sem0).wait()
        copy_in(g0 + 1, xbuf1, sem1).start()
        process(xbuf0, g0)
        copy_in(g0 + 1, xbuf1, sem1).wait()

        @pl.when(gg + 1 < NGRP // 2)
        def _():
            copy_in(g0 + 2, xbuf0, sem0).start()

        process(xbuf1, g0 + 1)
        return carry

    lax.fori_loop(0, NGRP // 2, group_body, 0)
    pltpu.sync_copy(outbuf, out_hbm.at[pl.ds(wid * RPW, RPW)])


@jax.jit
def kernel(input):
    x = input.astype(jnp.bfloat16).reshape(R * N)
    mesh = plsc.VectorSubcoreMesh(
        core_axis_name="c", subcore_axis_name="s",
        num_cores=NC, num_subcores=NS)
    out = pl.kernel(
        _kernel_body,
        out_type=jax.ShapeDtypeStruct((R,), jnp.float32),
        mesh=mesh,
        compiler_params=pltpu.CompilerParams(needs_layout_passes=False),
        scratch_types=[
            pltpu.VMEM((GROUP * N,), jnp.bfloat16),
            pltpu.VMEM((GROUP * N,), jnp.bfloat16),
            pltpu.VMEM((RPW,), jnp.float32),
            pltpu.SemaphoreType.DMA,
            pltpu.SemaphoreType.DMA,
        ],
    )(x)
    return out.reshape(B, C)
